# baseline (speedup 1.0000x reference)
import functools

import jax
import jax.numpy as jnp
from jax import lax
from jax.experimental import pallas as pl
from jax.experimental.pallas import tpu as pltpu
from jax.experimental.pallas import tpu_sc as plsc

_N = 10000
_E = 320000
_D = 128
_G = 64

_NC = 2
_NS = 16
_NW = _NC * _NS
_CH = 128
_NPH = 2
_CPP = 40
_EPT = _NPH * _CPP * _CH
_EPAD = _NW * _EPT
_NPADROW = 16
_NP = _N + _NPADROW
_ZCH = 200
_NZCH = _N // _ZCH
_RB = 10000
_GRID = _N // _RB

_mesh = plsc.VectorSubcoreMesh(
    core_axis_name="c", subcore_axis_name="s", num_cores=_NC, num_subcores=_NS
)


@functools.partial(
    pl.kernel,
    out_type=jax.ShapeDtypeStruct((_NC * _N,), jnp.float32),
    mesh=_mesh,
    scratch_types=[
        pltpu.VMEM((_NPH, _CPP, _CH), jnp.int32),
        pltpu.VMEM((_CH,), jnp.float32),
        pltpu.VMEM((2000,), jnp.float32),
        pltpu.VMEM_SHARED((_NP,), jnp.float32),
        pltpu.SemaphoreType.DMA,
    ],
)
def _deg_kernel(dst_hbm, deg_out, didx, ones_v, zbuf, acc, sem):
    c = lax.axis_index("c")
    s = lax.axis_index("s")
    wid = c * _NS + s

    for j in range(_CH // 16):
        ones_v[pl.ds(j * 16, 16)] = jnp.full((16,), 1.0, jnp.float32)

    @pl.when(s == 0)
    def _():
        for j in range(2000 // 16):
            zbuf[pl.ds(j * 16, 16)] = jnp.zeros((16,), jnp.float32)
        for k in range(_N // 2000):
            pltpu.sync_copy(zbuf, acc.at[pl.ds(k * 2000, 2000)])

    pltpu.sync_copy(dst_hbm.at[wid], didx)
    plsc.subcore_barrier()

    for p in range(_NPH):
        def body(i, carry):
            pltpu.async_copy(ones_v, acc.at[didx.at[p, i]], sem, add=True)
            return carry

        lax.fori_loop(0, _CPP, body, 0)

        def dbody(i, carry):
            pltpu.make_async_copy(ones_v, acc.at[didx.at[p, i]], sem).wait()
            return carry

        lax.fori_loop(0, _CPP, dbody, 0)
    plsc.subcore_barrier()

    @pl.when(s == 0)
    def _():
        for k in range(_N // 2000):
            pltpu.sync_copy(acc.at[pl.ds(k * 2000, 2000)], zbuf)
            pltpu.sync_copy(
                zbuf, deg_out.at[pl.ds(pl.multiple_of(c * _N + k * 2000, 8), 2000)]
            )


@functools.partial(
    pl.kernel,
    out_type=jax.ShapeDtypeStruct((_NC * _N, _D), jnp.float32),
    mesh=_mesh,
    scratch_types=[
        pltpu.VMEM((_CPP, _CH), jnp.int32),
        pltpu.VMEM((_CPP, _CH), jnp.int32),
        pltpu.VMEM((_CH, _D), jnp.float32),
        pltpu.VMEM((_CH, _D), jnp.float32),
        pltpu.VMEM_SHARED((_NP, _D), jnp.float32),
        pltpu.SemaphoreType.DMA,
        pltpu.SemaphoreType.DMA,
    ],
)
def _agg_kernel(g_hbm, src_hbm, dst_hbm, z_hbm, out_hbm,
                sidx, didx, rows0, rows1, acc, sem0, sem1):
    c = lax.axis_index("c")
    s = lax.axis_index("s")
    wid = c * _NS + s

    pltpu.sync_copy(src_hbm.at[wid, 0], sidx)
    pltpu.sync_copy(dst_hbm.at[wid, 0], didx)
    pltpu.async_copy(g_hbm.at[sidx.at[0]], rows0, sem0)

    for r in range(-(-_NZCH // _NS)):
        chunk = r * _NS + s

        @pl.when(chunk < _NZCH)
        def _():
            off = pl.multiple_of(chunk * _ZCH, 8)
            pltpu.sync_copy(z_hbm, acc.at[pl.ds(off, _ZCH), :])

    plsc.subcore_barrier()

    for p in range(_NPH):
        if p > 0:
            pltpu.sync_copy(src_hbm.at[wid, p], sidx)
            pltpu.sync_copy(dst_hbm.at[wid, p], didx)
            pltpu.async_copy(g_hbm.at[sidx.at[0]], rows0, sem0)

        def body(j, carry):
            i0 = 2 * j
            pltpu.async_copy(g_hbm.at[sidx.at[i0 + 1]], rows1, sem1)
            pltpu.make_async_copy(g_hbm.at[sidx.at[i0]], rows0, sem0).wait()
            pltpu.sync_copy(rows0, acc.at[didx.at[i0]], add=True)
            pltpu.async_copy(g_hbm.at[sidx.at[i0 + 2]], rows0, sem0)
            pltpu.make_async_copy(g_hbm.at[sidx.at[i0 + 1]], rows1, sem1).wait()
            pltpu.sync_copy(rows1, acc.at[didx.at[i0 + 1]], add=True)
            return carry

        lax.fori_loop(0, (_CPP - 2) // 2, body, 0)
        pltpu.async_copy(g_hbm.at[sidx.at[_CPP - 1]], rows1, sem1)
        pltpu.make_async_copy(g_hbm.at[sidx.at[_CPP - 2]], rows0, sem0).wait()
        pltpu.sync_copy(rows0, acc.at[didx.at[_CPP - 2]], add=True)
        pltpu.make_async_copy(g_hbm.at[sidx.at[_CPP - 1]], rows1, sem1).wait()
        pltpu.sync_copy(rows1, acc.at[didx.at[_CPP - 1]], add=True)

    plsc.subcore_barrier()

    for r in range(-(-_NZCH // _NS)):
        chunk = r * _NS + s

        @pl.when(chunk < _NZCH)
        def _():
            off = pl.multiple_of(chunk * _ZCH, 8)
            pltpu.sync_copy(
                acc.at[pl.ds(off, _ZCH), :],
                out_hbm.at[pl.ds(pl.multiple_of(c * _N + off, 8), _ZCH), :],
            )


def _dinv_of(d0, d1):
    deg = d0 + d1
    return jnp.where(deg > 0, lax.rsqrt(deg), 0.0)


def _fuse1_body(x_ref, w_ref, d0_ref, d1_ref, o_ref):
    dinv = _dinv_of(d0_ref[...], d1_ref[...])
    o_ref[...] = dinv * jnp.dot(
        x_ref[...], w_ref[...], preferred_element_type=jnp.float32
    )


def _fuse2_body(pa_ref, pb_ref, d0_ref, d1_ref, b_ref, w_ref, o_ref):
    dinv = _dinv_of(d0_ref[...], d1_ref[...])
    h = jnp.maximum(dinv * (pa_ref[...] + pb_ref[...]) + b_ref[...], 0.0)
    o_ref[...] = dinv * jnp.dot(h, w_ref[...], preferred_element_type=jnp.float32)


def _fuse3_body(pa_ref, pb_ref, d0_ref, d1_ref, b_ref, bt_ref, hw_ref, hb_ref,
                o_ref, sums, counts):
    i = pl.program_id(0)

    @pl.when(i == 0)
    def _():
        sums[...] = jnp.zeros_like(sums)
        counts[...] = jnp.zeros_like(counts)

    dinv = _dinv_of(d0_ref[...], d1_ref[...])
    h = jnp.maximum(dinv * (pa_ref[...] + pb_ref[...]) + b_ref[...], 0.0)
    gids = lax.broadcasted_iota(jnp.int32, (_RB, _G), 1)
    onehot = (bt_ref[...] == gids).astype(jnp.float32)
    sums[...] += lax.dot_general(
        onehot, h, (((0,), (0,)), ((), ())), preferred_element_type=jnp.float32
    )
    counts[...] += lax.dot_general(
        onehot, jnp.ones((_RB, 1), jnp.float32),
        (((0,), (0,)), ((), ())), preferred_element_type=jnp.float32
    )

    @pl.when(i == pl.num_programs(0) - 1)
    def _():
        emb = sums[...] / jnp.maximum(counts[...], 1.0)
        o_ref[...] = (
            jnp.dot(emb, hw_ref[...], preferred_element_type=jnp.float32)
            + hb_ref[...]
        )


_fuse1 = pl.pallas_call(
    _fuse1_body,
    grid=(_GRID,),
    in_specs=[
        pl.BlockSpec((_RB, _D), lambda i: (i, 0)),
        pl.BlockSpec((_D, _D), lambda i: (0, 0)),
        pl.BlockSpec((_RB, 1), lambda i: (i, 0)),
        pl.BlockSpec((_RB, 1), lambda i: (i + _GRID, 0)),
    ],
    out_specs=pl.BlockSpec((_RB, _D), lambda i: (i, 0)),
    out_shape=jax.ShapeDtypeStruct((_N, _D), jnp.float32),
)

_fuse2 = pl.pallas_call(
    _fuse2_body,
    grid=(_GRID,),
    in_specs=[
        pl.BlockSpec((_RB, _D), lambda i: (i, 0)),
        pl.BlockSpec((_RB, _D), lambda i: (i + _GRID, 0)),
        pl.BlockSpec((_RB, 1), lambda i: (i, 0)),
        pl.BlockSpec((_RB, 1), lambda i: (i + _GRID, 0)),
        pl.BlockSpec((1, _D), lambda i: (0, 0)),
        pl.BlockSpec((_D, _D), lambda i: (0, 0)),
    ],
    out_specs=pl.BlockSpec((_RB, _D), lambda i: (i, 0)),
    out_shape=jax.ShapeDtypeStruct((_N, _D), jnp.float32),
)

_fuse3 = pl.pallas_call(
    _fuse3_body,
    grid=(_GRID,),
    in_specs=[
        pl.BlockSpec((_RB, _D), lambda i: (i, 0)),
        pl.BlockSpec((_RB, _D), lambda i: (i + _GRID, 0)),
        pl.BlockSpec((_RB, 1), lambda i: (i, 0)),
        pl.BlockSpec((_RB, 1), lambda i: (i + _GRID, 0)),
        pl.BlockSpec((1, _D), lambda i: (0, 0)),
        pl.BlockSpec((_RB, 1), lambda i: (i, 0)),
        pl.BlockSpec((_D, 1), lambda i: (0, 0)),
        pl.BlockSpec((1, 1), lambda i: (0, 0)),
    ],
    out_specs=pl.BlockSpec((_G, 1), lambda i: (0, 0)),
    out_shape=jax.ShapeDtypeStruct((_G, 1), jnp.float32),
    scratch_shapes=[
        pltpu.VMEM((_G, _D), jnp.float32),
        pltpu.VMEM((_G, 1), jnp.float32),
    ],
)


def kernel(x, edge_index, batch, W1, b1, W2, b2, head_w, head_b):
    npad = _EPAD - _E
    dummy_src = jnp.arange(npad, dtype=jnp.int32) % _N
    dummy_dst = _N + (jnp.arange(npad, dtype=jnp.int32) % _NPADROW)
    src4 = jnp.concatenate([edge_index[0], dummy_src]).reshape(_NW, _NPH, _CPP, _CH)
    dst4 = jnp.concatenate([edge_index[1], dummy_dst]).reshape(_NW, _NPH, _CPP, _CH)
    zrows = jnp.zeros((_ZCH, _D), jnp.float32)

    degcol = _deg_kernel(dst4).reshape(_NC * _N, 1)
    g1 = _fuse1(x, W1, degcol, degcol)
    p1 = _agg_kernel(g1, src4, dst4, zrows)
    g2 = _fuse2(p1, p1, degcol, degcol, b1.reshape(1, _D), W2)
    p2 = _agg_kernel(g2, src4, dst4, zrows)
    return _fuse3(p2, p2, degcol, degcol, b2.reshape(1, _D),
                  batch.reshape(_N, 1), head_w, head_b.reshape(1, 1))


# speedup vs baseline: 1.0149x; 1.0149x over previous
import functools

import jax
import jax.numpy as jnp
from jax import lax
from jax.experimental import pallas as pl
from jax.experimental.pallas import tpu as pltpu
from jax.experimental.pallas import tpu_sc as plsc

_N = 10000
_E = 320000
_D = 128
_G = 64

_NC = 2
_NS = 16
_NW = _NC * _NS
_CH = 128
_NPH = 2
_CPP = 40
_EPT = _NPH * _CPP * _CH
_EPAD = _NW * _EPT
_NPADROW = 16
_NP = _N + _NPADROW
_ZCH = 200
_NZCH = _N // _ZCH
_RB = 5000
_GRID = _N // _RB

_mesh = plsc.VectorSubcoreMesh(
    core_axis_name="c", subcore_axis_name="s", num_cores=_NC, num_subcores=_NS
)


@functools.partial(
    pl.kernel,
    out_type=jax.ShapeDtypeStruct((_NC * _N,), jnp.float32),
    mesh=_mesh,
    scratch_types=[
        pltpu.VMEM((_NPH, _CPP, _CH), jnp.int32),
        pltpu.VMEM((_CH,), jnp.float32),
        pltpu.VMEM((2000,), jnp.float32),
        pltpu.VMEM_SHARED((_NP,), jnp.float32),
        pltpu.SemaphoreType.DMA,
    ],
)
def _deg_kernel(dst_hbm, deg_out, didx, ones_v, zbuf, acc, sem):
    c = lax.axis_index("c")
    s = lax.axis_index("s")
    wid = c * _NS + s

    for j in range(_CH // 16):
        ones_v[pl.ds(j * 16, 16)] = jnp.full((16,), 1.0, jnp.float32)

    @pl.when(s == 0)
    def _():
        for j in range(2000 // 16):
            zbuf[pl.ds(j * 16, 16)] = jnp.zeros((16,), jnp.float32)
        for k in range(_N // 2000):
            pltpu.sync_copy(zbuf, acc.at[pl.ds(k * 2000, 2000)])

    pltpu.sync_copy(dst_hbm.at[wid], didx)
    plsc.subcore_barrier()

    for p in range(_NPH):
        def body(i, carry):
            pltpu.async_copy(ones_v, acc.at[didx.at[p, i]], sem, add=True)
            return carry

        lax.fori_loop(0, _CPP, body, 0)

        def dbody(i, carry):
            pltpu.make_async_copy(ones_v, acc.at[didx.at[p, i]], sem).wait()
            return carry

        lax.fori_loop(0, _CPP, dbody, 0)
    plsc.subcore_barrier()

    @pl.when(s == 0)
    def _():
        for k in range(_N // 2000):
            pltpu.sync_copy(acc.at[pl.ds(k * 2000, 2000)], zbuf)
            pltpu.sync_copy(
                zbuf, deg_out.at[pl.ds(pl.multiple_of(c * _N + k * 2000, 8), 2000)]
            )


@functools.partial(
    pl.kernel,
    out_type=jax.ShapeDtypeStruct((_NC * _N, _D), jnp.float32),
    mesh=_mesh,
    scratch_types=[
        pltpu.VMEM((_CPP, _CH), jnp.int32),
        pltpu.VMEM((_CPP, _CH), jnp.int32),
        pltpu.VMEM((_CH, _D), jnp.float32),
        pltpu.VMEM((_CH, _D), jnp.float32),
        pltpu.VMEM_SHARED((_NP, _D), jnp.float32),
        pltpu.SemaphoreType.DMA,
        pltpu.SemaphoreType.DMA,
    ],
)
def _agg_kernel(g_hbm, src_hbm, dst_hbm, z_hbm, out_hbm,
                sidx, didx, rows0, rows1, acc, sem0, sem1):
    c = lax.axis_index("c")
    s = lax.axis_index("s")
    wid = c * _NS + s

    pltpu.sync_copy(src_hbm.at[wid, 0], sidx)
    pltpu.sync_copy(dst_hbm.at[wid, 0], didx)
    pltpu.async_copy(g_hbm.at[sidx.at[0]], rows0, sem0)

    for r in range(-(-_NZCH // _NS)):
        chunk = r * _NS + s

        @pl.when(chunk < _NZCH)
        def _():
            off = pl.multiple_of(chunk * _ZCH, 8)
            pltpu.sync_copy(z_hbm, acc.at[pl.ds(off, _ZCH), :])

    plsc.subcore_barrier()

    for p in range(_NPH):
        if p > 0:
            pltpu.sync_copy(src_hbm.at[wid, p], sidx)
            pltpu.sync_copy(dst_hbm.at[wid, p], didx)
            pltpu.async_copy(g_hbm.at[sidx.at[0]], rows0, sem0)

        def body(j, carry):
            i0 = 2 * j
            pltpu.async_copy(g_hbm.at[sidx.at[i0 + 1]], rows1, sem1)
            pltpu.make_async_copy(g_hbm.at[sidx.at[i0]], rows0, sem0).wait()
            pltpu.sync_copy(rows0, acc.at[didx.at[i0]], add=True)
            pltpu.async_copy(g_hbm.at[sidx.at[i0 + 2]], rows0, sem0)
            pltpu.make_async_copy(g_hbm.at[sidx.at[i0 + 1]], rows1, sem1).wait()
            pltpu.sync_copy(rows1, acc.at[didx.at[i0 + 1]], add=True)
            return carry

        lax.fori_loop(0, (_CPP - 2) // 2, body, 0)
        pltpu.async_copy(g_hbm.at[sidx.at[_CPP - 1]], rows1, sem1)
        pltpu.make_async_copy(g_hbm.at[sidx.at[_CPP - 2]], rows0, sem0).wait()
        pltpu.sync_copy(rows0, acc.at[didx.at[_CPP - 2]], add=True)
        pltpu.make_async_copy(g_hbm.at[sidx.at[_CPP - 1]], rows1, sem1).wait()
        pltpu.sync_copy(rows1, acc.at[didx.at[_CPP - 1]], add=True)

    plsc.subcore_barrier()

    for r in range(-(-_NZCH // _NS)):
        chunk = r * _NS + s

        @pl.when(chunk < _NZCH)
        def _():
            off = pl.multiple_of(chunk * _ZCH, 8)
            pltpu.sync_copy(
                acc.at[pl.ds(off, _ZCH), :],
                out_hbm.at[pl.ds(pl.multiple_of(c * _N + off, 8), _ZCH), :],
            )


def _dinv_of(d0, d1):
    deg = d0 + d1
    return jnp.where(deg > 0, lax.rsqrt(deg), 0.0)


def _fuse1_body(x_ref, w_ref, d0_ref, d1_ref, o_ref):
    dinv = _dinv_of(d0_ref[...], d1_ref[...])
    o_ref[...] = dinv * jnp.dot(
        x_ref[...], w_ref[...], preferred_element_type=jnp.float32
    )


def _fuse2_body(pa_ref, pb_ref, d0_ref, d1_ref, b_ref, w_ref, o_ref):
    dinv = _dinv_of(d0_ref[...], d1_ref[...])
    h = jnp.maximum(dinv * (pa_ref[...] + pb_ref[...]) + b_ref[...], 0.0)
    o_ref[...] = dinv * jnp.dot(h, w_ref[...], preferred_element_type=jnp.float32)


def _fuse3_body(pa_ref, pb_ref, d0_ref, d1_ref, b_ref, bt_ref, hw_ref, hb_ref,
                o_ref, sums, counts):
    i = pl.program_id(0)

    @pl.when(i == 0)
    def _():
        sums[...] = jnp.zeros_like(sums)
        counts[...] = jnp.zeros_like(counts)

    dinv = _dinv_of(d0_ref[...], d1_ref[...])
    h = jnp.maximum(dinv * (pa_ref[...] + pb_ref[...]) + b_ref[...], 0.0)
    gids = lax.broadcasted_iota(jnp.int32, (_RB, _G), 1)
    onehot = (bt_ref[...] == gids).astype(jnp.float32)
    sums[...] += lax.dot_general(
        onehot, h, (((0,), (0,)), ((), ())), preferred_element_type=jnp.float32
    )
    counts[...] += lax.dot_general(
        onehot, jnp.ones((_RB, 1), jnp.float32),
        (((0,), (0,)), ((), ())), preferred_element_type=jnp.float32
    )

    @pl.when(i == pl.num_programs(0) - 1)
    def _():
        emb = sums[...] / jnp.maximum(counts[...], 1.0)
        o_ref[...] = (
            jnp.dot(emb, hw_ref[...], preferred_element_type=jnp.float32)
            + hb_ref[...]
        )


_fuse1 = pl.pallas_call(
    _fuse1_body,
    grid=(_GRID,),
    in_specs=[
        pl.BlockSpec((_RB, _D), lambda i: (i, 0)),
        pl.BlockSpec((_D, _D), lambda i: (0, 0)),
        pl.BlockSpec((_RB, 1), lambda i: (i, 0)),
        pl.BlockSpec((_RB, 1), lambda i: (i + _GRID, 0)),
    ],
    out_specs=pl.BlockSpec((_RB, _D), lambda i: (i, 0)),
    out_shape=jax.ShapeDtypeStruct((_N, _D), jnp.float32),
)

_fuse2 = pl.pallas_call(
    _fuse2_body,
    grid=(_GRID,),
    in_specs=[
        pl.BlockSpec((_RB, _D), lambda i: (i, 0)),
        pl.BlockSpec((_RB, _D), lambda i: (i + _GRID, 0)),
        pl.BlockSpec((_RB, 1), lambda i: (i, 0)),
        pl.BlockSpec((_RB, 1), lambda i: (i + _GRID, 0)),
        pl.BlockSpec((1, _D), lambda i: (0, 0)),
        pl.BlockSpec((_D, _D), lambda i: (0, 0)),
    ],
    out_specs=pl.BlockSpec((_RB, _D), lambda i: (i, 0)),
    out_shape=jax.ShapeDtypeStruct((_N, _D), jnp.float32),
)

_fuse3 = pl.pallas_call(
    _fuse3_body,
    grid=(_GRID,),
    in_specs=[
        pl.BlockSpec((_RB, _D), lambda i: (i, 0)),
        pl.BlockSpec((_RB, _D), lambda i: (i + _GRID, 0)),
        pl.BlockSpec((_RB, 1), lambda i: (i, 0)),
        pl.BlockSpec((_RB, 1), lambda i: (i + _GRID, 0)),
        pl.BlockSpec((1, _D), lambda i: (0, 0)),
        pl.BlockSpec((_RB, 1), lambda i: (i, 0)),
        pl.BlockSpec((_D, 1), lambda i: (0, 0)),
        pl.BlockSpec((1, 1), lambda i: (0, 0)),
    ],
    out_specs=pl.BlockSpec((_G, 1), lambda i: (0, 0)),
    out_shape=jax.ShapeDtypeStruct((_G, 1), jnp.float32),
    scratch_shapes=[
        pltpu.VMEM((_G, _D), jnp.float32),
        pltpu.VMEM((_G, 1), jnp.float32),
    ],
)


def kernel(x, edge_index, batch, W1, b1, W2, b2, head_w, head_b):
    npad = _EPAD - _E
    dummy_src = jnp.arange(npad, dtype=jnp.int32) % _N
    dummy_dst = _N + (jnp.arange(npad, dtype=jnp.int32) % _NPADROW)
    src4 = jnp.concatenate([edge_index[0], dummy_src]).reshape(_NW, _NPH, _CPP, _CH)
    dst4 = jnp.concatenate([edge_index[1], dummy_dst]).reshape(_NW, _NPH, _CPP, _CH)
    zrows = jnp.zeros((_ZCH, _D), jnp.float32)

    degcol = _deg_kernel(dst4).reshape(_NC * _N, 1)
    g1 = _fuse1(x, W1, degcol, degcol)
    p1 = _agg_kernel(g1, src4, dst4, zrows)
    g2 = _fuse2(p1, p1, degcol, degcol, b1.reshape(1, _D), W2)
    p2 = _agg_kernel(g2, src4, dst4, zrows)
    return _fuse3(p2, p2, degcol, degcol, b2.reshape(1, _D),
                  batch.reshape(_N, 1), head_w, head_b.reshape(1, 1))
